# baseline (device time: 701696 ns/iter reference)
import jax
import jax.numpy as jnp
from jax import lax
from jax.experimental import pallas as pl
from jax.experimental.pallas import tpu as pltpu

N_DEV = 8
N_PASS = 2


def _gelu(y):
    c = 0.7978845608028654
    return 0.5 * y * (1.0 + jnp.tanh(c * (y + 0.044715 * y * y * y)))


def kernel(x, w_mat):
    m_tot, _ = x.shape
    _, n = w_mat.shape
    m_per = m_tot // N_DEV
    n_q = n // 4

    def body(x_ref, w_ref, out_ref, sendR, sendL, recvR, recvL,
             send_semR, send_semL, recv_semsR, recv_semsL,
             out_semR, out_semL, creditR, creditL):
        my = lax.axis_index("i")
        left = lax.rem(my + N_DEV - 1, N_DEV)
        right = lax.rem(my + 1, N_DEV)

        barrier = pltpu.get_barrier_semaphore()
        for nbr in (left, right):
            pl.semaphore_signal(barrier, inc=1, device_id=(nbr,),
                                device_id_type=pl.DeviceIdType.MESH)
        pl.semaphore_wait(barrier, 2)

        def partial(c, q):
            xc = x_ref[pl.ds(c * m_per, m_per), :]
            wq = w_ref[:, q * n_q:(q + 1) * n_q]
            return jnp.dot(xc, wq, preferred_element_type=jnp.float32)

        def rdma(src, dst_slots, slot, send_sem, recv_sems, to):
            return pltpu.make_async_remote_copy(
                src_ref=src,
                dst_ref=dst_slots.at[slot],
                send_sem=send_sem,
                recv_sem=recv_sems.at[slot],
                device_id=(to,),
                device_id_type=pl.DeviceIdType.MESH,
            )

        rings = (
            (sendR, recvR, send_semR, recv_semsR, right, left, creditR, out_semR),
            (sendL, recvL, send_semL, recv_semsL, left, right, creditL, out_semL),
        )
        pending = [None, None]
        out_copies = []

        n_steps = N_DEV - 1
        for k in range(N_PASS * n_steps):
            jq, t = k // n_steps, k % n_steps
            cR = lax.rem(my + N_DEV - 1 - t, N_DEV)
            cL = lax.rem(my + 1 + t, N_DEV)
            parts = (partial(cR, jq), partial(cL, 2 + jq))
            if out_copies:
                for cp in out_copies:
                    cp.wait()
                out_copies = []
                for r in range(2):
                    _sb, _rs, _ss, _rse, _to, upstream, credit, _os = rings[r]
                    pl.semaphore_signal(
                        credit, inc=1, device_id=(upstream,),
                        device_id_type=pl.DeviceIdType.MESH)
            for r in ((0, 1), (1, 0))[k % 2]:
                sbuf, rslots, ssem, rsems, to, upstream, credit, _osem = rings[r]
                p = parts[r]
                if t == 0:
                    if k > 0:
                        pending[r].wait()
                    sbuf[...] = p
                else:
                    pending[r].wait()
                    sbuf[...] = rslots[(k - 1) % 2] + p
                    if k <= 12:
                        pl.semaphore_signal(
                            credit, inc=1, device_id=(upstream,),
                            device_id_type=pl.DeviceIdType.MESH)
                if k >= 2:
                    pl.semaphore_wait(credit, 1)
                pending[r] = rdma(sbuf, rslots, k % 2, ssem, rsems, to)
                pending[r].start()

            if t == 0 and k > 0:
                slot = (k - 1) % 2
                eparts = (partial(my, jq - 1), partial(my, 2 + jq - 1))
                for r in range(2):
                    sbuf, rslots, _ssem, _rsems, _to, _up, _credit, osem = rings[r]
                    q = (jq - 1, 2 + jq - 1)[r]
                    rslots[slot, :, :] = _gelu(rslots[slot] + eparts[r])
                    cp = pltpu.make_async_copy(
                        rslots.at[slot], out_ref.at[:, q * n_q:(q + 1) * n_q],
                        osem)
                    cp.start()
                    out_copies.append(cp)

        k_last = N_PASS * n_steps - 1
        eparts = (partial(my, N_PASS - 1), partial(my, 2 + N_PASS - 1))
        copies = []
        for r in range(2):
            sbuf, rslots, _ssem, _rsems, _to, _up, _credit, osem = rings[r]
            q = (N_PASS - 1, 2 + N_PASS - 1)[r]
            pending[r].wait()
            sbuf[...] = _gelu(rslots[k_last % 2] + eparts[r])
            cp = pltpu.make_async_copy(
                sbuf, out_ref.at[:, q * n_q:(q + 1) * n_q], osem)
            cp.start()
            copies.append(cp)
        for cp in copies:
            cp.wait()

    return pl.pallas_call(
        body,
        out_shape=jax.ShapeDtypeStruct((m_per, n), jnp.float32),
        in_specs=[
            pl.BlockSpec(memory_space=pltpu.VMEM),
            pl.BlockSpec(memory_space=pltpu.VMEM),
        ],
        out_specs=pl.BlockSpec(memory_space=pl.ANY),
        scratch_shapes=[
            pltpu.VMEM((m_per, n_q), jnp.float32),
            pltpu.VMEM((m_per, n_q), jnp.float32),
            pltpu.VMEM((2, m_per, n_q), jnp.float32),
            pltpu.VMEM((2, m_per, n_q), jnp.float32),
            pltpu.SemaphoreType.DMA,
            pltpu.SemaphoreType.DMA,
            pltpu.SemaphoreType.DMA((2,)),
            pltpu.SemaphoreType.DMA((2,)),
            pltpu.SemaphoreType.DMA,
            pltpu.SemaphoreType.DMA,
            pltpu.SemaphoreType.REGULAR,
            pltpu.SemaphoreType.REGULAR,
        ],
        compiler_params=pltpu.CompilerParams(
            collective_id=0,
            vmem_limit_bytes=60 * 1024 * 1024,
        ),
    )(x, w_mat)


# device time: 697786 ns/iter; 1.0056x vs baseline; 1.0056x over previous
import jax
import jax.numpy as jnp
from jax import lax
from jax.experimental import pallas as pl
from jax.experimental.pallas import tpu as pltpu

N_DEV = 8
N_PASS = 2
N_SUB = 2


def _gelu(y):
    c = 0.7978845608028654
    return 0.5 * y * (1.0 + jnp.tanh(c * (y + 0.044715 * y * y * y)))


def kernel(x, w_mat):
    m_tot, _ = x.shape
    _, n = w_mat.shape
    m_per = m_tot // N_DEV
    n_q = n // 4
    n_s = n_q // N_SUB

    grant_max = (N_PASS * (N_DEV - 1) - 2) * N_SUB - 1

    def body(x_ref, w_ref, out_ref, sendR, sendL, recvR, recvL,
             send_semsR, send_semsL, recv_semsR, recv_semsL,
             out_semsR, out_semsL, creditR, creditL):
        my = lax.axis_index("i")
        left = lax.rem(my + N_DEV - 1, N_DEV)
        right = lax.rem(my + 1, N_DEV)

        barrier = pltpu.get_barrier_semaphore()
        for nbr in (left, right):
            pl.semaphore_signal(barrier, inc=1, device_id=(nbr,),
                                device_id_type=pl.DeviceIdType.MESH)
        pl.semaphore_wait(barrier, 2)

        def partial(c, q):
            xc = x_ref[pl.ds(c * m_per, m_per), :]
            wq = w_ref[:, q * n_q:(q + 1) * n_q]
            return jnp.dot(xc, wq, preferred_element_type=jnp.float32)

        def sub_rdma(sbuf, rslots, slot, s, ssems, rsems, to):
            return pltpu.make_async_remote_copy(
                src_ref=sbuf.at[s],
                dst_ref=rslots.at[slot, s],
                send_sem=ssems.at[s],
                recv_sem=rsems.at[slot, s],
                device_id=(to,),
                device_id_type=pl.DeviceIdType.MESH,
            )

        rings = (
            (sendR, recvR, send_semsR, recv_semsR, right, left, creditR,
             out_semsR),
            (sendL, recvL, send_semsL, recv_semsL, left, right, creditL,
             out_semsL),
        )
        pending = [[None] * N_SUB, [None] * N_SUB]
        n_steps = N_DEV - 1

        for jq in range(N_PASS):
            for t in range(n_steps):
                k = jq * n_steps + t
                cR = lax.rem(my + N_DEV - 1 - t, N_DEV)
                cL = lax.rem(my + 1 + t, N_DEV)
                parts = (partial(cR, jq), partial(cL, 2 + jq))
                for r in ((0, 1), (1, 0))[k % 2]:
                    sbuf, rslots, ssems, rsems, to, upstream, credit, _os = \
                        rings[r]
                    p = parts[r]
                    slot, pslot = k % 2, (k - 1) % 2
                    for s in range(N_SUB):
                        j = N_SUB * k + s
                        ps = p[:, s * n_s:(s + 1) * n_s]
                        if t == 0:
                            sbuf[s, :, :] = ps
                        else:
                            pending[r][s].wait()
                            sbuf[s, :, :] = rslots[pslot, s] + ps
                            if j - 2 <= grant_max:
                                pl.semaphore_signal(
                                    credit, inc=1, device_id=(upstream,),
                                    device_id_type=pl.DeviceIdType.MESH)
                        if j >= 2 * N_SUB:
                            pl.semaphore_wait(credit, 1)
                        pending[r][s] = sub_rdma(
                            sbuf, rslots, slot, s, ssems, rsems, to)
                        pending[r][s].start()

            slot = (jq * n_steps + n_steps - 1) % 2
            eparts = (partial(my, jq), partial(my, 2 + jq))
            copies = []
            for r in range(2):
                sbuf, rslots, _ss, _rs, _to, upstream, credit, osems = rings[r]
                q = (jq, 2 + jq)[r]
                for s in range(N_SUB):
                    pending[r][s].wait()
                    pending[r][s] = None
                    sbuf[s, :, :] = _gelu(
                        rslots[slot, s] + eparts[r][:, s * n_s:(s + 1) * n_s])
                    if jq < N_PASS - 1:
                        pl.semaphore_signal(
                            credit, inc=1, device_id=(upstream,),
                            device_id_type=pl.DeviceIdType.MESH)
                    cp = pltpu.make_async_copy(
                        sbuf.at[s],
                        out_ref.at[:, q * n_q + s * n_s:
                                   q * n_q + (s + 1) * n_s],
                        osems.at[s])
                    cp.start()
                    copies.append(cp)
            for cp in copies:
                cp.wait()

    return pl.pallas_call(
        body,
        out_shape=jax.ShapeDtypeStruct((m_per, n), jnp.float32),
        in_specs=[
            pl.BlockSpec(memory_space=pltpu.VMEM),
            pl.BlockSpec(memory_space=pltpu.VMEM),
        ],
        out_specs=pl.BlockSpec(memory_space=pl.ANY),
        scratch_shapes=[
            pltpu.VMEM((N_SUB, m_per, n_s), jnp.float32),
            pltpu.VMEM((N_SUB, m_per, n_s), jnp.float32),
            pltpu.VMEM((2, N_SUB, m_per, n_s), jnp.float32),
            pltpu.VMEM((2, N_SUB, m_per, n_s), jnp.float32),
            pltpu.SemaphoreType.DMA((N_SUB,)),
            pltpu.SemaphoreType.DMA((N_SUB,)),
            pltpu.SemaphoreType.DMA((2, N_SUB)),
            pltpu.SemaphoreType.DMA((2, N_SUB)),
            pltpu.SemaphoreType.DMA((N_SUB,)),
            pltpu.SemaphoreType.DMA((N_SUB,)),
            pltpu.SemaphoreType.REGULAR,
            pltpu.SemaphoreType.REGULAR,
        ],
        compiler_params=pltpu.CompilerParams(
            collective_id=0,
            vmem_limit_bytes=60 * 1024 * 1024,
        ),
    )(x, w_mat)
